# baseline (speedup 1.0000x reference)
import jax
import jax.numpy as jnp
import numpy as np
from jax.experimental import pallas as pl
from jax.experimental.pallas import tpu as pltpu

N = 8192
NSUB = 8
SUB = N // NSUB
K = int(SUB * 0.05)
ITERS = 5
BI = 4096
BJ = 1024
KPAD = 64
NSEL = NSUB * K
NGRP = NSEL // 8

_MININT = np.int32(-2**31)


def _orderkey(x):
    f = jax.lax.bitcast_convert_type(x, jnp.int32)
    return jnp.where(f >= 0, f, jnp.bitwise_not(f ^ _MININT))


def _topk_mask(xa):
    key = _orderkey(xa)
    p = jnp.zeros((NSUB, 1), jnp.int32)
    for b in range(31, -1, -1):
        trial = p | np.int32(1 << b) if b < 31 else p | _MININT
        th = trial ^ _MININT
        cnt = jnp.sum((key >= th).astype(jnp.int32), axis=1, keepdims=True)
        p = jnp.where(cnt >= K, trial, p)
    vk = p ^ _MININT
    gt = key > vk
    eq = key == vk
    need = K - jnp.sum(gt.astype(jnp.int32), axis=1, keepdims=True)
    idx = jax.lax.broadcasted_iota(jnp.int32, (NSUB, SUB), 1)
    lo = jnp.zeros((NSUB, 1), jnp.int32)
    hi = jnp.full((NSUB, 1), SUB, jnp.int32)
    for _ in range(11):
        mid = (lo + hi) // 2
        c = jnp.sum((eq & (idx < mid)).astype(jnp.int32), axis=1, keepdims=True)
        ok = c >= need
        hi = jnp.where(ok, mid, hi)
        lo = jnp.where(ok, lo, mid + 1)
    sel = gt | (eq & (idx < hi))
    return sel.astype(jnp.float32)


def _activation(xa, noise):
    scale = (1e-10 + jnp.max(xa) - jnp.min(xa)) / 100000.0
    return _topk_mask(xa + scale * noise)


def _body_a(h0_ref, w_ref, wt_ref, x1_ref, xacc_ref, hq_ref):
    ih = pl.program_id(0)
    jb = pl.program_id(1)

    @pl.when((ih == 0) & (jb == 0))
    def _():
        hq_ref[...] = h0_ref[...].astype(jnp.bfloat16).astype(jnp.float32)

    hseg = hq_ref[pl.ds(jb, 1), :]

    part = None
    for ch in range(BJ // 128):
        wb = w_ref[:, 128 * ch:128 * (ch + 1)].astype(jnp.bfloat16)
        wf = wb.astype(jnp.float32)
        wt_ref[pl.ds(128 * ch, 128), :] = wf.T.astype(jnp.bfloat16)
        p = wf * hseg[:, 128 * ch:128 * (ch + 1)]
        part = p if part is None else part + p
    xp = jnp.sum(part.reshape(BI // 128, 128, 128), axis=-1)

    @pl.when(jb == 0)
    def _():
        xacc_ref[...] = xp

    @pl.when(jb != 0)
    def _():
        xacc_ref[...] += xp

    @pl.when(jb == N // BJ - 1)
    def _():
        x1_ref[pl.ds(ih * (BI // 128), BI // 128), :] = xacc_ref[...]


NBUF = 8


def _body_b(x1_ref, noise_ref, tri_ref, wt_ref, out_ref,
            xa_ref, idx_ref, idxs_ref, buf_ref, sem, csem):
    it = pl.program_id(0)

    @pl.when(it == 0)
    def _():
        xa_ref[...] = x1_ref[...].reshape(NSUB, SUB)

    mask = _activation(xa_ref[...], noise_ref[pl.ds(it, 1)][0])

    rank = jax.lax.dot_general(
        mask.astype(jnp.bfloat16), tri_ref[...],
        (((1,), (0,)), ((), ())), preferred_element_type=jnp.float32)
    ranki = rank.astype(jnp.int32)
    kio = jax.lax.broadcasted_iota(jnp.int32, (1, 1, KPAD), 2)
    cnt = jnp.sum((ranki[:, :, None] <= kio).astype(jnp.int32), axis=1)
    roff = jax.lax.broadcasted_iota(jnp.int32, (NSUB, KPAD), 0) * SUB
    idx_ref[...] = cnt + roff

    pltpu.make_async_copy(idx_ref, idxs_ref, csem).start()
    pltpu.make_async_copy(idx_ref, idxs_ref, csem).wait()

    def issue(k, slot):
        r = k // K
        g = k - r * K
        j = idxs_ref[r, g]
        pltpu.make_async_copy(wt_ref.at[pl.ds(j, 1)],
                              buf_ref.at[pl.ds(slot, 1)],
                              sem.at[slot]).start()

    for k in range(NBUF):
        issue(k, k)

    def gloop(k, xsum):
        slot = jax.lax.rem(k, NBUF)
        pltpu.make_async_copy(wt_ref.at[pl.ds(0, 1)],
                              buf_ref.at[pl.ds(slot, 1)],
                              sem.at[slot]).wait()
        row = buf_ref[pl.ds(slot, 1)][0].astype(jnp.float32)

        @pl.when(k + NBUF < NSEL)
        def _():
            issue(k + NBUF, slot)

        return xsum + row

    x_new = jax.lax.fori_loop(
        0, NSEL, gloop, jnp.zeros((NSUB, SUB), jnp.float32))
    xa_ref[...] = x_new

    @pl.when(it == ITERS - 2)
    def _():
        out_ref[...] = _activation(x_new, noise_ref[pl.ds(ITERS - 1, 1)][0])


@jax.jit
def kernel(h_0, w):
    keys = []
    key = jax.random.key(42)
    for _ in range(ITERS):
        key, sub = jax.random.split(key)
        keys.append(sub)
    noise = jnp.stack(
        [jax.random.normal(k, (N,), jnp.float32) for k in keys]
    ).reshape(ITERS, NSUB, SUB)
    h0 = h_0.reshape(NSUB, SUB)
    tri = jnp.triu(jnp.ones((SUB, SUB), jnp.bfloat16))

    wt, x1 = pl.pallas_call(
        _body_a,
        grid=(N // BI, N // BJ),
        in_specs=[
            pl.BlockSpec((NSUB, SUB), lambda ih, jb: (0, 0)),
            pl.BlockSpec((BI, BJ), lambda ih, jb: (ih, jb)),
        ],
        out_specs=[
            pl.BlockSpec((BJ, BI), lambda ih, jb: (jb, ih)),
            pl.BlockSpec((N // 128, 128), lambda ih, jb: (0, 0)),
        ],
        out_shape=[
            jax.ShapeDtypeStruct((N, N), jnp.bfloat16),
            jax.ShapeDtypeStruct((N // 128, 128), jnp.float32),
        ],
        scratch_shapes=[
            pltpu.VMEM((BI // 128, 128), jnp.float32),
            pltpu.VMEM((NSUB, SUB), jnp.float32),
        ],
    )(h0, w)

    out = pl.pallas_call(
        _body_b,
        grid=(ITERS - 1,),
        in_specs=[
            pl.BlockSpec((N // 128, 128), lambda it: (0, 0)),
            pl.BlockSpec((ITERS, NSUB, SUB), lambda it: (0, 0, 0)),
            pl.BlockSpec((SUB, SUB), lambda it: (0, 0)),
            pl.BlockSpec(memory_space=pl.ANY),
        ],
        out_specs=pl.BlockSpec((NSUB, SUB), lambda it: (0, 0)),
        out_shape=jax.ShapeDtypeStruct((NSUB, SUB), jnp.float32),
        scratch_shapes=[
            pltpu.VMEM((NSUB, SUB), jnp.float32),
            pltpu.VMEM((NSUB, KPAD), jnp.int32),
            pltpu.SMEM((NSUB, KPAD), jnp.int32),
            pltpu.VMEM((NBUF, NSUB, SUB), jnp.bfloat16),
            pltpu.SemaphoreType.DMA((NBUF,)),
            pltpu.SemaphoreType.DMA,
        ],
    )(x1, noise, tri, wt.reshape(N, NSUB, SUB))
    return out.reshape(N)


# speedup vs baseline: 1.3953x; 1.3953x over previous
import jax
import jax.numpy as jnp
import numpy as np
from jax.experimental import pallas as pl
from jax.experimental.pallas import tpu as pltpu

N = 8192
NSUB = 8
SUB = N // NSUB
K = int(SUB * 0.05)
ITERS = 5
BI = 4096
BJ = 1024
KPAD = 64
NSEL = NSUB * K
NGRP = NSEL // 8

_MININT = np.int32(-2**31)


def _orderkey(x):
    f = jax.lax.bitcast_convert_type(x, jnp.int32)
    return jnp.where(f >= 0, f, jnp.bitwise_not(f ^ _MININT))


def _topk_mask(xa):
    key = _orderkey(xa)
    p = jnp.zeros((NSUB, 1), jnp.int32)
    for b in range(31, -1, -1):
        trial = p | np.int32(1 << b) if b < 31 else p | _MININT
        th = trial ^ _MININT
        cnt = jnp.sum((key >= th).astype(jnp.int32), axis=1, keepdims=True)
        p = jnp.where(cnt >= K, trial, p)
    vk = p ^ _MININT
    gt = key > vk
    eq = key == vk
    need = K - jnp.sum(gt.astype(jnp.int32), axis=1, keepdims=True)
    idx = jax.lax.broadcasted_iota(jnp.int32, (NSUB, SUB), 1)
    lo = jnp.zeros((NSUB, 1), jnp.int32)
    hi = jnp.full((NSUB, 1), SUB, jnp.int32)
    for _ in range(11):
        mid = (lo + hi) // 2
        c = jnp.sum((eq & (idx < mid)).astype(jnp.int32), axis=1, keepdims=True)
        ok = c >= need
        hi = jnp.where(ok, mid, hi)
        lo = jnp.where(ok, lo, mid + 1)
    sel = gt | (eq & (idx < hi))
    return sel.astype(jnp.float32)


def _activation(xa, noise):
    scale = (1e-10 + jnp.max(xa) - jnp.min(xa)) / 100000.0
    return _topk_mask(xa + scale * noise)


def _body_a(h0_ref, w_ref, wt_ref, x1_ref, xacc_ref, hq_ref):
    ih = pl.program_id(0)
    jb = pl.program_id(1)

    @pl.when((ih == 0) & (jb == 0))
    def _():
        hq_ref[...] = h0_ref[...].astype(jnp.bfloat16).astype(jnp.float32)

    hseg = hq_ref[pl.ds(jb, 1), :]

    part = None
    for ch in range(BJ // 128):
        wb = w_ref[:, 128 * ch:128 * (ch + 1)].astype(jnp.bfloat16)
        wf = wb.astype(jnp.float32)
        wt_ref[pl.ds(128 * ch, 128), :] = wf.T.astype(jnp.bfloat16)
        p = wf * hseg[:, 128 * ch:128 * (ch + 1)]
        part = p if part is None else part + p
    xp = jnp.sum(part.reshape(BI // 128, 128, 128), axis=-1)

    @pl.when(jb == 0)
    def _():
        xacc_ref[...] = xp

    @pl.when(jb != 0)
    def _():
        xacc_ref[...] += xp

    @pl.when(jb == N // BJ - 1)
    def _():
        x1_ref[pl.ds(ih * (BI // 128), BI // 128), :] = xacc_ref[...]


NBUF = 64


def _body_b(x1_ref, noise_ref, tri_ref, wt_ref, out_ref,
            xa_ref, idx_ref, idxs_ref, buf_ref, sem, csem):
    it = pl.program_id(0)

    @pl.when(it == 0)
    def _():
        xa_ref[...] = x1_ref[...].reshape(NSUB, SUB)

    mask = _activation(xa_ref[...], noise_ref[pl.ds(it, 1)][0])

    rank = jax.lax.dot_general(
        mask.astype(jnp.bfloat16), tri_ref[...],
        (((1,), (0,)), ((), ())), preferred_element_type=jnp.float32)
    ranki = rank.astype(jnp.int32)
    kio = jax.lax.broadcasted_iota(jnp.int32, (1, 1, KPAD), 2)
    cnt = jnp.sum((ranki[:, :, None] <= kio).astype(jnp.int32), axis=1)
    roff = jax.lax.broadcasted_iota(jnp.int32, (NSUB, KPAD), 0) * SUB
    idx_ref[...] = cnt + roff

    pltpu.make_async_copy(idx_ref, idxs_ref, csem).start()
    pltpu.make_async_copy(idx_ref, idxs_ref, csem).wait()

    def issue(k, slot):
        r = k // K
        g = k - r * K
        j = idxs_ref[r, g]
        pltpu.make_async_copy(wt_ref.at[pl.ds(j, 1)],
                              buf_ref.at[pl.ds(slot, 1)],
                              sem.at[slot]).start()

    for k in range(NBUF):
        issue(k, k)

    def gloop(k, xsum):
        slot = jax.lax.rem(k, NBUF)
        pltpu.make_async_copy(wt_ref.at[pl.ds(0, 1)],
                              buf_ref.at[pl.ds(slot, 1)],
                              sem.at[slot]).wait()
        row = buf_ref[pl.ds(slot, 1)][0].astype(jnp.float32)

        @pl.when(k + NBUF < NSEL)
        def _():
            issue(k + NBUF, slot)

        return xsum + row

    x_new = jax.lax.fori_loop(
        0, NSEL, gloop, jnp.zeros((NSUB, SUB), jnp.float32))
    xa_ref[...] = x_new

    @pl.when(it == ITERS - 2)
    def _():
        out_ref[...] = _activation(x_new, noise_ref[pl.ds(ITERS - 1, 1)][0])


@jax.jit
def kernel(h_0, w):
    keys = []
    key = jax.random.key(42)
    for _ in range(ITERS):
        key, sub = jax.random.split(key)
        keys.append(sub)
    noise = jnp.stack(
        [jax.random.normal(k, (N,), jnp.float32) for k in keys]
    ).reshape(ITERS, NSUB, SUB)
    h0 = h_0.reshape(NSUB, SUB)
    tri = jnp.triu(jnp.ones((SUB, SUB), jnp.bfloat16))

    wt, x1 = pl.pallas_call(
        _body_a,
        grid=(N // BI, N // BJ),
        in_specs=[
            pl.BlockSpec((NSUB, SUB), lambda ih, jb: (0, 0)),
            pl.BlockSpec((BI, BJ), lambda ih, jb: (ih, jb)),
        ],
        out_specs=[
            pl.BlockSpec((BJ, BI), lambda ih, jb: (jb, ih)),
            pl.BlockSpec((N // 128, 128), lambda ih, jb: (0, 0)),
        ],
        out_shape=[
            jax.ShapeDtypeStruct((N, N), jnp.bfloat16),
            jax.ShapeDtypeStruct((N // 128, 128), jnp.float32),
        ],
        scratch_shapes=[
            pltpu.VMEM((BI // 128, 128), jnp.float32),
            pltpu.VMEM((NSUB, SUB), jnp.float32),
        ],
    )(h0, w)

    out = pl.pallas_call(
        _body_b,
        grid=(ITERS - 1,),
        in_specs=[
            pl.BlockSpec((N // 128, 128), lambda it: (0, 0)),
            pl.BlockSpec((ITERS, NSUB, SUB), lambda it: (0, 0, 0)),
            pl.BlockSpec((SUB, SUB), lambda it: (0, 0)),
            pl.BlockSpec(memory_space=pl.ANY),
        ],
        out_specs=pl.BlockSpec((NSUB, SUB), lambda it: (0, 0)),
        out_shape=jax.ShapeDtypeStruct((NSUB, SUB), jnp.float32),
        scratch_shapes=[
            pltpu.VMEM((NSUB, SUB), jnp.float32),
            pltpu.VMEM((NSUB, KPAD), jnp.int32),
            pltpu.SMEM((NSUB, KPAD), jnp.int32),
            pltpu.VMEM((NBUF, NSUB, SUB), jnp.bfloat16),
            pltpu.SemaphoreType.DMA((NBUF,)),
            pltpu.SemaphoreType.DMA,
        ],
    )(x1, noise, tri, wt.reshape(N, NSUB, SUB))
    return out.reshape(N)
